# initial kernel scaffold (unmeasured)
import jax
import jax.numpy as jnp
from jax import lax
from jax.experimental import pallas as pl
from jax.experimental.pallas import tpu as pltpu

N_DEV = 4
M_BLK = 2048
K_SH = 2048
N_OUT = 4096
BN = 1024
BK = 2048


def _a2a_body(x_ref, out_ref, send_sems, recv_sems, local_sem):
    my = lax.axis_index("i")

    barrier = pltpu.get_barrier_semaphore()
    for d in range(1, N_DEV):
        pl.semaphore_signal(
            barrier, inc=1,
            device_id=((my + d) % N_DEV,),
            device_id_type=pl.DeviceIdType.MESH,
        )
    pl.semaphore_wait(barrier, N_DEV - 1)

    local = pltpu.make_async_copy(
        x_ref.at[pl.ds(my * M_BLK, M_BLK), :],
        out_ref.at[:, pl.ds(my * K_SH, K_SH)],
        local_sem,
    )
    local.start()

    rdmas = []
    for d in range(1, N_DEV):
        t = (my + d) % N_DEV
        r = pltpu.make_async_remote_copy(
            src_ref=x_ref.at[pl.ds(t * M_BLK, M_BLK), :],
            dst_ref=out_ref.at[:, pl.ds(my * K_SH, K_SH)],
            send_sem=send_sems.at[d - 1],
            recv_sem=recv_sems.at[d - 1],
            device_id=(t,),
            device_id_type=pl.DeviceIdType.MESH,
        )
        r.start()
        rdmas.append(r)

    local.wait()
    for r in rdmas:
        r.wait()


def _gemm_body(x_ref, w_ref, out_ref, acc_ref):
    k = pl.program_id(1)
    nk = pl.num_programs(1)

    @pl.when(k == 0)
    def _():
        acc_ref[...] = jnp.zeros_like(acc_ref)

    acc_ref[...] += jnp.dot(
        x_ref[...], w_ref[...], preferred_element_type=jnp.float32
    )

    @pl.when(k == nk - 1)
    def _():
        a = acc_ref[...]
        out_ref[...] = a * jax.nn.sigmoid(a)


def kernel(x, w_mat):
    xt = pl.pallas_call(
        _a2a_body,
        out_shape=jax.ShapeDtypeStruct((M_BLK, N_DEV * K_SH), jnp.bfloat16),
        in_specs=[pl.BlockSpec(memory_space=pltpu.ANY)],
        out_specs=pl.BlockSpec(memory_space=pltpu.VMEM),
        scratch_shapes=[
            pltpu.SemaphoreType.DMA((N_DEV - 1,)),
            pltpu.SemaphoreType.DMA((N_DEV - 1,)),
            pltpu.SemaphoreType.DMA,
        ],
        compiler_params=pltpu.CompilerParams(collective_id=0),
    )(x)

    grid = (N_OUT // BN, (N_DEV * K_SH) // BK)
    y = pl.pallas_call(
        _gemm_body,
        grid=grid,
        in_specs=[
            pl.BlockSpec((M_BLK, BK), lambda n, k: (0, k)),
            pl.BlockSpec((BK, BN), lambda n, k: (k, n)),
        ],
        out_specs=pl.BlockSpec((M_BLK, BN), lambda n, k: (0, n)),
        out_shape=jax.ShapeDtypeStruct((M_BLK, N_OUT), jnp.float32),
        scratch_shapes=[pltpu.VMEM((M_BLK, BN), jnp.float32)],
        compiler_params=pltpu.CompilerParams(
            dimension_semantics=("parallel", "arbitrary"),
        ),
    )(xt, w_mat)
    return y


# baseline (device time: 297043 ns/iter reference)
import jax
import jax.numpy as jnp
from jax import lax
from jax.experimental import pallas as pl
from jax.experimental.pallas import tpu as pltpu

try:
    jax.config.update("jax_compilation_cache_dir", "/tmp/jax_comp_cache")
    jax.config.update("jax_persistent_cache_min_compile_time_secs", 1.0)
    jax.config.update("jax_persistent_cache_min_entry_size_bytes", 0)
except Exception:
    pass

N_DEV = 4
M_BLK = 2048
K_SH = 2048
N_OUT = 4096
BN = 512
NH = 2
H_ROWS = M_BLK // NH
N_NT = N_OUT // BN


def _fused_body(x_ref, w_ref, out_ref, own_buf, recv_buf, w_buf,
                send_sems, recv_sems, w_sems, local_sem):
    my = lax.axis_index("i")

    barrier = pltpu.get_barrier_semaphore()
    for d in range(1, N_DEV):
        pl.semaphore_signal(
            barrier, inc=1,
            device_id=((my + d) % N_DEV,),
            device_id_type=pl.DeviceIdType.MESH,
        )
    pl.semaphore_wait(barrier, N_DEV - 1)

    local = pltpu.make_async_copy(
        x_ref.at[pl.ds(my * M_BLK, M_BLK), :], own_buf, local_sem)
    local.start()

    rdmas = {}
    for d in range(1, N_DEV):
        t = (my + d) % N_DEV
        for h in range(NH):
            g = (d - 1) * NH + h
            r = pltpu.make_async_remote_copy(
                src_ref=x_ref.at[pl.ds(t * M_BLK + h * H_ROWS, H_ROWS), :],
                dst_ref=recv_buf.at[g],
                send_sem=send_sems.at[g],
                recv_sem=recv_sems.at[g],
                device_id=(t,),
                device_id_type=pl.DeviceIdType.MESH,
            )
            r.start()
            rdmas[(d, h)] = r

    steps = []
    for nt in range(N_NT):
        steps.append((None, None, my, nt))
    for h in range(NH):
        for d in (1, 3, 2):
            s = (my - d) % N_DEV
            for nt in range(N_NT):
                steps.append((d, h, s, nt))

    w_descs = []
    for k, (d, h, s, nt) in enumerate(steps):
        w_descs.append(pltpu.make_async_copy(
            w_ref.at[pl.ds(s * K_SH, K_SH), pl.ds(nt * BN, BN)],
            w_buf.at[k % 2], w_sems.at[k % 2]))

    w_descs[0].start()
    for k, (d, h, s, nt) in enumerate(steps):
        w_descs[k].wait()
        if k + 1 < len(steps):
            w_descs[k + 1].start()
        if nt == 0:
            if d is None:
                local.wait()
            else:
                rdmas[(d, h)].wait_recv()
        wt = w_buf[k % 2].astype(jnp.bfloat16)
        cols = slice(nt * BN, (nt + 1) * BN)
        if d is None:
            acc = jnp.dot(own_buf[...], wt,
                          preferred_element_type=jnp.float32)
            out_ref[:, cols] = acc.astype(jnp.bfloat16)
        else:
            rows = slice(h * H_ROWS, (h + 1) * H_ROWS)
            a = recv_buf[(d - 1) * NH + h]
            acc = jnp.dot(a, wt, preferred_element_type=jnp.float32)
            tot = out_ref[rows, cols].astype(jnp.float32) + acc
            if d == 2:
                tot = tot * jax.nn.sigmoid(tot)
            out_ref[rows, cols] = tot.astype(jnp.bfloat16)

    for r in rdmas.values():
        r.wait_send()


def kernel(x, w_mat):
    x = x.astype(jnp.bfloat16)
    return pl.pallas_call(
        _fused_body,
        out_shape=jax.ShapeDtypeStruct((M_BLK, N_OUT), jnp.bfloat16),
        in_specs=[
            pl.BlockSpec(memory_space=pl.ANY),
            pl.BlockSpec(memory_space=pl.ANY),
        ],
        out_specs=pl.BlockSpec(memory_space=pltpu.MemorySpace.VMEM),
        scratch_shapes=[
            pltpu.VMEM((M_BLK, K_SH), jnp.bfloat16),
            pltpu.VMEM(((N_DEV - 1) * NH, H_ROWS, K_SH), jnp.bfloat16),
            pltpu.VMEM((2, K_SH, BN), jnp.float32),
            pltpu.SemaphoreType.DMA(((N_DEV - 1) * NH,)),
            pltpu.SemaphoreType.DMA(((N_DEV - 1) * NH,)),
            pltpu.SemaphoreType.DMA((2,)),
            pltpu.SemaphoreType.DMA,
        ],
        compiler_params=pltpu.CompilerParams(
            collective_id=0,
            vmem_limit_bytes=100 * 1024 * 1024,
        ),
    )(x, w_mat)
